# baseline (speedup 1.0000x reference)
import jax
import jax.numpy as jnp
from jax.experimental import pallas as pl
from jax.experimental.pallas import tpu as pltpu

N_ROWS = 64
N_COLS = 32768
GR = 8
NGROUP = N_ROWS // GR
NBUF_IN = 3
NBUF_OUT = 2


def _hardmax_pipeline(x_hbm, o_hbm, xbuf, obuf, rsem, wsem):
    def rd(g, slot):
        return pltpu.make_async_copy(
            x_hbm.at[pl.ds(g * GR, GR), :], xbuf.at[slot], rsem.at[slot])

    def wr(g, slot):
        return pltpu.make_async_copy(
            obuf.at[slot], o_hbm.at[pl.ds(g * GR, GR), :], wsem.at[slot])

    for s in range(min(NBUF_IN, NGROUP)):
        rd(s, s).start()

    for g in range(NGROUP):
        rd(g, g % NBUF_IN).wait()
        xb = xbuf[g % NBUF_IN]
        idx = jnp.argmax(xb, axis=1, keepdims=True)
        if g >= NBUF_OUT:
            wr(g - NBUF_OUT, g % NBUF_OUT).wait()
        iota = jax.lax.broadcasted_iota(jnp.int32, (GR, N_COLS), 1)
        obuf[g % NBUF_OUT] = (iota == idx).astype(jnp.int32)
        wr(g, g % NBUF_OUT).start()
        if g + NBUF_IN < NGROUP:
            rd(g + NBUF_IN, g % NBUF_IN).start()

    for g in range(max(NGROUP - NBUF_OUT, 0), NGROUP):
        wr(g, g % NBUF_OUT).wait()


def kernel(x):
    return pl.pallas_call(
        _hardmax_pipeline,
        in_specs=[pl.BlockSpec(memory_space=pl.ANY)],
        out_specs=pl.BlockSpec(memory_space=pl.ANY),
        out_shape=jax.ShapeDtypeStruct((N_ROWS, N_COLS), jnp.int32),
        scratch_shapes=[
            pltpu.VMEM((NBUF_IN, GR, N_COLS), jnp.float32),
            pltpu.VMEM((NBUF_OUT, GR, N_COLS), jnp.int32),
            pltpu.SemaphoreType.DMA((NBUF_IN,)),
            pltpu.SemaphoreType.DMA((NBUF_OUT,)),
        ],
    )(x)


# speedup vs baseline: 1.1866x; 1.1866x over previous
import jax
import jax.numpy as jnp
from jax.experimental import pallas as pl
from jax.experimental.pallas import tpu as pltpu

N_ROWS = 64
N_COLS = 32768
GR = 16
NGROUP = N_ROWS // GR
NBUF_IN = 3
NBUF_OUT = 2


def _hardmax_pipeline(x_hbm, o_hbm, xbuf, obuf, rsem, wsem):
    def rd(g, slot):
        return pltpu.make_async_copy(
            x_hbm.at[pl.ds(g * GR, GR), :], xbuf.at[slot], rsem.at[slot])

    def wr(g, slot):
        return pltpu.make_async_copy(
            obuf.at[slot], o_hbm.at[pl.ds(g * GR, GR), :], wsem.at[slot])

    for s in range(min(NBUF_IN, NGROUP)):
        rd(s, s).start()

    for g in range(NGROUP):
        rd(g, g % NBUF_IN).wait()
        xb = xbuf[g % NBUF_IN]
        idx = jnp.argmax(xb, axis=1, keepdims=True)
        if g >= NBUF_OUT:
            wr(g - NBUF_OUT, g % NBUF_OUT).wait()
        iota = jax.lax.broadcasted_iota(jnp.int32, (GR, N_COLS), 1)
        obuf[g % NBUF_OUT] = (iota == idx).astype(jnp.int32)
        wr(g, g % NBUF_OUT).start()
        if g + NBUF_IN < NGROUP:
            rd(g + NBUF_IN, g % NBUF_IN).start()

    for g in range(max(NGROUP - NBUF_OUT, 0), NGROUP):
        wr(g, g % NBUF_OUT).wait()


def kernel(x):
    return pl.pallas_call(
        _hardmax_pipeline,
        in_specs=[pl.BlockSpec(memory_space=pl.ANY)],
        out_specs=pl.BlockSpec(memory_space=pl.ANY),
        out_shape=jax.ShapeDtypeStruct((N_ROWS, N_COLS), jnp.int32),
        scratch_shapes=[
            pltpu.VMEM((NBUF_IN, GR, N_COLS), jnp.float32),
            pltpu.VMEM((NBUF_OUT, GR, N_COLS), jnp.int32),
            pltpu.SemaphoreType.DMA((NBUF_IN,)),
            pltpu.SemaphoreType.DMA((NBUF_OUT,)),
        ],
    )(x)


# speedup vs baseline: 1.2374x; 1.0428x over previous
import jax
import jax.numpy as jnp
from jax.experimental import pallas as pl
from jax.experimental.pallas import tpu as pltpu

N_ROWS = 64
N_COLS = 32768
GR = 16
NGROUP = N_ROWS // GR
NBUF_IN = 4
NBUF_OUT = 2


def _hardmax_pipeline(x_hbm, o_hbm, xbuf, obuf, rsem, wsem):
    def rd(g, slot):
        return pltpu.make_async_copy(
            x_hbm.at[pl.ds(g * GR, GR), :], xbuf.at[slot], rsem.at[slot])

    def wr(g, slot):
        return pltpu.make_async_copy(
            obuf.at[slot], o_hbm.at[pl.ds(g * GR, GR), :], wsem.at[slot])

    for s in range(min(NBUF_IN, NGROUP)):
        rd(s, s).start()

    for g in range(NGROUP):
        rd(g, g % NBUF_IN).wait()
        xb = xbuf[g % NBUF_IN]
        idx = jnp.argmax(xb, axis=1, keepdims=True)
        if g >= NBUF_OUT:
            wr(g - NBUF_OUT, g % NBUF_OUT).wait()
        iota = jax.lax.broadcasted_iota(jnp.int32, (GR, N_COLS), 1)
        obuf[g % NBUF_OUT] = (iota == idx).astype(jnp.int32)
        wr(g, g % NBUF_OUT).start()
        if g + NBUF_IN < NGROUP:
            rd(g + NBUF_IN, g % NBUF_IN).start()

    for g in range(max(NGROUP - NBUF_OUT, 0), NGROUP):
        wr(g, g % NBUF_OUT).wait()


def kernel(x):
    return pl.pallas_call(
        _hardmax_pipeline,
        in_specs=[pl.BlockSpec(memory_space=pl.ANY)],
        out_specs=pl.BlockSpec(memory_space=pl.ANY),
        out_shape=jax.ShapeDtypeStruct((N_ROWS, N_COLS), jnp.int32),
        scratch_shapes=[
            pltpu.VMEM((NBUF_IN, GR, N_COLS), jnp.float32),
            pltpu.VMEM((NBUF_OUT, GR, N_COLS), jnp.int32),
            pltpu.SemaphoreType.DMA((NBUF_IN,)),
            pltpu.SemaphoreType.DMA((NBUF_OUT,)),
        ],
    )(x)


# speedup vs baseline: 1.2826x; 1.0365x over previous
import jax
import jax.numpy as jnp
from jax.experimental import pallas as pl
from jax.experimental.pallas import tpu as pltpu

N_ROWS = 64
N_COLS = 32768
GROUPS = ((0, 16), (16, 16), (32, 16), (48, 8), (56, 8))
NG = len(GROUPS)


def _hardmax_pipeline(x_hbm, o_hbm, xbuf, obuf, rsem, wsem):
    def rd(g):
        lo, n = GROUPS[g]
        return pltpu.make_async_copy(
            x_hbm.at[pl.ds(lo, n), :], xbuf.at[pl.ds(lo, n), :], rsem.at[g])

    def wr(g):
        lo, n = GROUPS[g]
        return pltpu.make_async_copy(
            obuf.at[pl.ds(lo, n), :], o_hbm.at[pl.ds(lo, n), :], wsem.at[g])

    for g in range(NG):
        rd(g).start()

    for g in range(NG):
        lo, n = GROUPS[g]
        rd(g).wait()
        xb = xbuf[pl.ds(lo, n), :]
        idx = jnp.argmax(xb, axis=1, keepdims=True)
        iota = jax.lax.broadcasted_iota(jnp.int32, (n, N_COLS), 1)
        obuf[pl.ds(lo, n), :] = (iota == idx).astype(jnp.int32)
        wr(g).start()

    for g in range(NG):
        wr(g).wait()


def kernel(x):
    return pl.pallas_call(
        _hardmax_pipeline,
        in_specs=[pl.BlockSpec(memory_space=pl.ANY)],
        out_specs=pl.BlockSpec(memory_space=pl.ANY),
        out_shape=jax.ShapeDtypeStruct((N_ROWS, N_COLS), jnp.int32),
        scratch_shapes=[
            pltpu.VMEM((N_ROWS, N_COLS), jnp.float32),
            pltpu.VMEM((N_ROWS, N_COLS), jnp.int32),
            pltpu.SemaphoreType.DMA((NG,)),
            pltpu.SemaphoreType.DMA((NG,)),
        ],
    )(x)
